# baseline (speedup 1.0000x reference)
import functools

import jax
import jax.numpy as jnp
from jax import lax
from jax.experimental import pallas as pl
from jax.experimental.pallas import tpu as pltpu
from jax.experimental.pallas import tpu_sc as plsc

Qn = 1024
Dn = 512
Kn = 100000
TKn = 100
KB = 1024
KPAD = 100352

NW = 32
QT = Qn // NW
CH = 12544
NCH = KPAD // CH
CAND = 12544
OUTW = 128
NEG_INF = float("-inf")
I32_MAX = 2147483647


def _fusion_body(img_ref, desc_ref, w_ref, b_ref, q_ref):
    concat = jnp.concatenate([img_ref[...], desc_ref[...]], axis=-1)
    q_ref[...] = jnp.tanh(jnp.dot(concat, w_ref[...]) + b_ref[...])


def _scores_body(q_ref, keys_ref, knorm_ref, out_ref):
    kn = keys_ref[...] / knorm_ref[...]
    s = lax.dot_general(q_ref[...], kn, (((1,), (1,)), ((), ())))
    pid = pl.program_id(0)
    col = pid * KB + lax.broadcasted_iota(jnp.int32, s.shape, 1)
    out_ref[...] = jnp.where(col < Kn, s, -jnp.inf)


def _lane():
    return lax.iota(jnp.int32, 16)


def _ku_of(s):
    bits = lax.bitcast_convert_type(s, jnp.int32)
    sign = lax.shift_right_arithmetic(bits, 31)
    kui = bits ^ (sign | jnp.int32(-(2 ** 31)))
    return lax.bitcast_convert_type(kui, jnp.uint32)


def _pop1(mask):
    return plsc.all_reduce_population_count(mask)[0]


def _topk_body(scores, outv, outi, buf, buf2, cnds, cndi, cnds2, cndi2, hist,
               topv, topi, finv, fini, outvb, outib, sem_a, sem_b):
    cid = lax.axis_index("c")
    sid = lax.axis_index("s")
    wid = sid * 2 + cid

    ones16 = jnp.full((16,), 1, jnp.int32)
    zeros16 = jnp.zeros((16,), jnp.int32)
    neginf16 = jnp.full((16,), NEG_INF, jnp.float32)
    lane = _lane()

    def zero_hist():
        def zstep(i, _):
            hist[pl.ds(i * 16, 16)] = zeros16
            return 0
        lax.fori_loop(0, 256, zstep, 0)

    def walk(above0):
        def wstep(i, carry):
            above, bsel, found = carry
            d = 255 - i
            h = hist[pl.ds(d * 16, 16)]
            hs = jnp.sum(h)
            tot = above + hs
            hit = tot >= TKn
            take = jnp.logical_and(jnp.logical_not(found), hit)
            bsel = jnp.where(take, d, bsel)
            found_new = jnp.logical_or(found, hit)
            above = jnp.where(found_new, above, tot)
            return above, bsel, found_new
        above, bsel, _ = lax.fori_loop(
            0, 256, wstep, (above0, jnp.int32(0), jnp.bool_(False)))
        return above, bsel

    def spec_pass(row, t_ku):
        bufs = (buf, buf2)
        sems = (sem_a, sem_b)
        cps = {0: pltpu.async_copy(scores.at[row, pl.ds(0, CH)], buf, sem_a)}
        carry = (jnp.int32(0), jnp.bool_(False))
        for c in range(NCH):
            cur = bufs[c % 2]
            cps[c % 2].wait()
            if c + 1 < NCH:
                cps[(c + 1) % 2] = pltpu.async_copy(
                    scores.at[row, pl.ds((c + 1) * CH, CH)],
                    bufs[(c + 1) % 2], sems[(c + 1) % 2])

            def step(i, carry, cur=cur, c=c):
                coff, dropped = carry
                s = cur[pl.ds(i * 16, 16)]
                ku = _ku_of(s)
                m = ku >= t_ku

                def hit(coff=coff, dropped=dropped, s=s, m=m, i=i, c=c):
                    idx = (c * CH + i * 16) + lane
                    can = coff <= CAND - 16
                    mm = jnp.logical_and(m, can)
                    plsc.store_compressed(
                        cnds.at[pl.ds(coff, 16)], s, mask=mm)
                    plsc.store_compressed(
                        cndi.at[pl.ds(coff, 16)], idx, mask=mm)
                    nc = coff + _pop1(mm)
                    nd = jnp.logical_or(dropped, _pop1(m) > _pop1(mm))
                    return nc, nd

                def miss(coff=coff, dropped=dropped):
                    return coff, dropped

                return lax.cond(jnp.any(m), hit, miss)
            carry = lax.fori_loop(0, CH // 16, step, carry)
        return carry

    def hist_row(row, shift, prefix, use_prefix):
        def chunk(c, _):
            pltpu.sync_copy(scores.at[row, pl.ds(c * CH, CH)], buf)
            def step(i, _):
                s = buf[pl.ds(i * 16, 16)]
                ku = _ku_of(s)
                dig = lax.shift_right_logical(ku, jnp.uint32(shift))
                dig = lax.bitcast_convert_type(dig, jnp.int32) & 0xFF
                slot = dig * 16 + lane
                if use_prefix:
                    hi = lax.shift_right_logical(ku, jnp.uint32(shift + 8))
                    m = hi == prefix
                    plsc.addupdate_scatter(hist, [slot], ones16, mask=m)
                else:
                    plsc.addupdate_scatter(hist, [slot], ones16)
                return 0
            lax.fori_loop(0, CH // 16, step, 0)
            return 0
        lax.fori_loop(0, NCH, chunk, 0)

    def compact_row(row, shift, pasm):
        def chunk(c, carry):
            pltpu.sync_copy(scores.at[row, pl.ds(c * CH, CH)], buf)
            def step(i, carry):
                topoff, coff, dropped = carry
                s = buf[pl.ds(i * 16, 16)]
                idx = (c * CH + i * 16) + lane
                ku = _ku_of(s)
                kus = lax.shift_right_logical(ku, jnp.uint32(shift))
                gt = kus > pasm
                eq = kus == pasm
                plsc.store_compressed(topv.at[pl.ds(topoff, 16)], s, mask=gt)
                plsc.store_compressed(topi.at[pl.ds(topoff, 16)], idx, mask=gt)
                topoff = topoff + _pop1(gt)
                can = coff <= CAND - 16
                eqm = jnp.logical_and(eq, can)
                plsc.store_compressed(cnds.at[pl.ds(coff, 16)], s, mask=eqm)
                plsc.store_compressed(cndi.at[pl.ds(coff, 16)], idx, mask=eqm)
                coff = coff + _pop1(eqm)
                dropped = jnp.logical_or(
                    dropped, _pop1(eq) > _pop1(eqm))
                return topoff, coff, dropped
            return lax.fori_loop(0, CH // 16, step, carry)
        return lax.fori_loop(
            0, NCH, chunk,
            (jnp.int32(0), jnp.int32(0), jnp.bool_(False)))

    def hist_cand(shift, coff):
        def step(i, _):
            valid = lane < (coff - i * 16)
            s = cnds[pl.ds(i * 16, 16)]
            ku = _ku_of(s)
            dig = lax.shift_right_logical(ku, jnp.uint32(shift))
            dig = lax.bitcast_convert_type(dig, jnp.int32) & 0xFF
            slot = dig * 16 + lane
            plsc.addupdate_scatter(hist, [slot], ones16, mask=valid)
            return 0
        n16 = (coff + 15) // 16
        lax.fori_loop(0, n16, step, 0)

    def compact_cand(shift, pasm, topoff0, coff0):
        def step(i, carry):
            topoff, coff = carry
            valid = lane < (coff0 - i * 16)
            s = cnds[pl.ds(i * 16, 16)]
            idx = cndi[pl.ds(i * 16, 16)]
            ku = _ku_of(s)
            kus = lax.shift_right_logical(ku, jnp.uint32(shift))
            gt = jnp.logical_and(valid, kus > pasm)
            eq = jnp.logical_and(valid, kus == pasm)
            plsc.store_compressed(topv.at[pl.ds(topoff, 16)], s, mask=gt)
            plsc.store_compressed(topi.at[pl.ds(topoff, 16)], idx, mask=gt)
            topoff = topoff + _pop1(gt)
            plsc.store_compressed(cnds2.at[pl.ds(coff, 16)], s, mask=eq)
            plsc.store_compressed(cndi2.at[pl.ds(coff, 16)], idx, mask=eq)
            coff = coff + _pop1(eq)
            return topoff, coff
        n16 = (coff0 + 15) // 16
        topoff, coff = lax.fori_loop(0, n16, step, (topoff0, jnp.int32(0)))
        def cstep(i, _):
            cnds[pl.ds(i * 16, 16)] = cnds2[pl.ds(i * 16, 16)]
            cndi[pl.ds(i * 16, 16)] = cndi2[pl.ds(i * 16, 16)]
            return 0
        lax.fori_loop(0, (coff + 15) // 16, cstep, 0)
        return topoff, coff

    def do_row(r, t_ku):
        row = wid * QT + r

        scoff, sdrop = spec_pass(row, t_ku)
        ok = jnp.logical_and(scoff >= TKn, jnp.logical_not(sdrop))

        zero_hist()

        def h_ok(scoff=scoff):
            hist_cand(24, scoff)
            return 0

        def h_fb(row=row):
            hist_row(row, 24, jnp.uint32(0), False)
            return 0

        lax.cond(ok, h_ok, h_fb)
        above, b = walk(jnp.int32(0))
        prefix = lax.convert_element_type(b, jnp.uint32)

        def c_ok(scoff=scoff, prefix=prefix):
            to, co = compact_cand(24, prefix, jnp.int32(0), scoff)
            return to, co, jnp.bool_(True)

        def c_fb(row=row, prefix=prefix):
            to, co, drp = compact_row(row, 24, prefix)
            return to, co, jnp.logical_not(drp)

        topoff, coff, compacted = lax.cond(ok, c_ok, c_fb)

        for lvl in (1, 2, 3):
            shift = 24 - 8 * lvl
            zero_hist()

            def hc(coff=coff, shift=shift):
                hist_cand(shift, coff)
                return 0

            def hr(row=row, shift=shift, prefix=prefix):
                hist_row(row, shift, prefix, True)
                return 0

            lax.cond(compacted, hc, hr)
            above, b = walk(above)
            pasm = (prefix << jnp.uint32(8)) | lax.convert_element_type(
                b, jnp.uint32)

            def cc(topoff=topoff, coff=coff, shift=shift, pasm=pasm):
                to, co = compact_cand(shift, pasm, topoff, coff)
                return to, co, jnp.bool_(True)

            def cr(row=row, shift=shift, pasm=pasm, lvl=lvl):
                to, co, drp = compact_row(row, shift, pasm)
                if lvl == 3:
                    return to, co, jnp.bool_(True)
                return to, co, jnp.logical_not(drp)

            topoff, coff, compacted = lax.cond(compacted, cc, cr)
            prefix = pasm

        def fstep(v, _):
            valid = lane < (topoff - v * 16)
            tv = topv[pl.ds(v * 16, 16)]
            ti = topi[pl.ds(v * 16, 16)]
            finv[pl.ds(v * 16, 16)] = jnp.where(valid, tv, neginf16)
            fini[pl.ds(v * 16, 16)] = jnp.where(valid, ti, zeros16)
            return 0
        lax.fori_loop(0, OUTW // 16, fstep, 0)

        def estep(v, off):
            valid = lane < (coff - v * 16)
            can = off <= OUTW - 16
            m = jnp.logical_and(valid, can)
            s = cnds[pl.ds(v * 16, 16)]
            idx = cndi[pl.ds(v * 16, 16)]
            plsc.store_compressed(finv.at[pl.ds(off, 16)], s, mask=m)
            plsc.store_compressed(fini.at[pl.ds(off, 16)], idx, mask=m)
            return off + _pop1(m)
        lax.fori_loop(0, OUTW // 16, estep, topoff)

        vs = [finv[pl.ds(v * 16, 16)] for v in range(OUTW // 16)]
        iss = [fini[pl.ds(v * 16, 16)] for v in range(OUTW // 16)]

        def ostep(ov, carry):
            vs, iss = carry

            def sstep(j, carry):
                vs, iss, ovec, oidx = carry
                vs = list(vs)
                mv = vs[0]
                for v in range(1, len(vs)):
                    mv = jnp.maximum(mv, vs[v])
                m = jnp.max(mv)
                iv = jnp.where(vs[0] == m, iss[0], I32_MAX)
                for v in range(1, len(vs)):
                    iv = jnp.minimum(
                        iv, jnp.where(vs[v] == m, iss[v], I32_MAX))
                sel = jnp.min(iv)
                hit = lane == j
                ovec = jnp.where(hit, m, ovec)
                oidx = jnp.where(hit, sel, oidx)
                for v in range(len(vs)):
                    rm = jnp.logical_and(vs[v] == m, iss[v] == sel)
                    vs[v] = jnp.where(rm, NEG_INF, vs[v])
                return tuple(vs), iss, ovec, oidx

            vs, iss, ovec, oidx = lax.fori_loop(
                0, 16, sstep,
                (vs, iss, jnp.zeros((16,), jnp.float32), zeros16))
            outvb[pl.ds(ov * 16, 16)] = ovec
            outib[pl.ds(ov * 16, 16)] = oidx
            return vs, iss
        lax.fori_loop(0, (TKn + 15) // 16, ostep, (tuple(vs), tuple(iss)))

        pltpu.sync_copy(outvb, outv.at[row])
        pltpu.sync_copy(outib, outi.at[row])
        last = outvb[pl.ds(((TKn - 1) // 16) * 16, 16)]
        return _ku_of(last)[(TKn - 1) % 16]

    lax.fori_loop(0, QT, do_row, jnp.uint32(0))


@functools.partial(jax.jit, static_argnames=())
def _topk_sc(scores):
    mesh = plsc.VectorSubcoreMesh(core_axis_name="c", subcore_axis_name="s")
    f = pl.kernel(
        _topk_body,
        out_type=[
            jax.ShapeDtypeStruct((Qn, OUTW), jnp.float32),
            jax.ShapeDtypeStruct((Qn, OUTW), jnp.int32),
        ],
        mesh=mesh,
        compiler_params=pltpu.CompilerParams(needs_layout_passes=False),
        scratch_types=[
            pltpu.VMEM((CH,), jnp.float32),
            pltpu.VMEM((CH,), jnp.float32),
            pltpu.VMEM((CAND,), jnp.float32),
            pltpu.VMEM((CAND,), jnp.int32),
            pltpu.VMEM((CAND,), jnp.float32),
            pltpu.VMEM((CAND,), jnp.int32),
            pltpu.VMEM((4096,), jnp.int32),
            pltpu.VMEM((OUTW,), jnp.float32),
            pltpu.VMEM((OUTW,), jnp.int32),
            pltpu.VMEM((OUTW,), jnp.float32),
            pltpu.VMEM((OUTW,), jnp.int32),
            pltpu.VMEM((OUTW,), jnp.float32),
            pltpu.VMEM((OUTW,), jnp.int32),
            pltpu.SemaphoreType.DMA,
            pltpu.SemaphoreType.DMA,
        ],
    )
    return f(scores)


def kernel(input_image_emb, input_description_emb, W_fusion, b_fusion, keys):
    fused = pl.pallas_call(
        _fusion_body,
        out_shape=jax.ShapeDtypeStruct((Qn, Dn), jnp.float32),
    )(input_image_emb, input_description_emb, W_fusion,
      b_fusion.reshape(1, Dn))

    q = fused / (jnp.linalg.norm(fused, axis=-1, keepdims=True) + 1e-12)
    knorm = jnp.linalg.norm(keys, axis=-1, keepdims=True) + 1e-12

    keys_pad = jnp.pad(keys, ((0, KPAD - Kn), (0, 0)))
    knorm_pad = jnp.pad(knorm, ((0, KPAD - Kn), (0, 0)), constant_values=1.0)
    scores = pl.pallas_call(
        _scores_body,
        grid=(KPAD // KB,),
        in_specs=[
            pl.BlockSpec((Qn, Dn), lambda i: (0, 0)),
            pl.BlockSpec((KB, Dn), lambda i: (i, 0)),
            pl.BlockSpec((KB, 1), lambda i: (i, 0)),
        ],
        out_specs=pl.BlockSpec((Qn, KB), lambda i: (0, i)),
        out_shape=jax.ShapeDtypeStruct((Qn, KPAD), jnp.float32),
    )(q, keys_pad, knorm_pad)

    tv, ti = _topk_sc(scores)
    return tv[:, :TKn], ti[:, :TKn]


# speedup vs baseline: 3.2610x; 3.2610x over previous
import functools

import jax
import jax.numpy as jnp
from jax import lax
from jax.experimental import pallas as pl
from jax.experimental.pallas import tpu as pltpu
from jax.experimental.pallas import tpu_sc as plsc

Qn = 1024
Dn = 512
Kn = 100000
TKn = 100
KB = 1024
KPAD = 100352

NW = 32
QT = Qn // NW
CH = 12544
NCH = KPAD // CH
CAND = 12544
OUTW = 128
NEG_INF = float("-inf")
I32_MAX = 2147483647


def _fusion_body(img_ref, desc_ref, w_ref, b_ref, q_ref):
    concat = jnp.concatenate([img_ref[...], desc_ref[...]], axis=-1)
    q_ref[...] = jnp.tanh(jnp.dot(concat, w_ref[...]) + b_ref[...])


def _scores_body(q_ref, keys_ref, knorm_ref, out_ref):
    kn = keys_ref[...] / knorm_ref[...]
    s = lax.dot_general(q_ref[...], kn, (((1,), (1,)), ((), ())))
    pid = pl.program_id(0)
    col = pid * KB + lax.broadcasted_iota(jnp.int32, s.shape, 1)
    out_ref[...] = jnp.where(col < Kn, s, -jnp.inf)


def _lane():
    return lax.iota(jnp.int32, 16)


def _ku_of(s):
    bits = lax.bitcast_convert_type(s, jnp.int32)
    sign = lax.shift_right_arithmetic(bits, 31)
    kui = bits ^ (sign | jnp.int32(-(2 ** 31)))
    return lax.bitcast_convert_type(kui, jnp.uint32)


def _pop1(mask):
    return plsc.all_reduce_population_count(mask)[0]


def _topk_body(scores, outv, outi, buf, buf2, cnds, cndi, cnds2, cndi2, hist,
               topv, topi, finv, fini, outvb, outib, sem_a, sem_b):
    cid = lax.axis_index("c")
    sid = lax.axis_index("s")
    wid = sid * 2 + cid

    ones16 = jnp.full((16,), 1, jnp.int32)
    zeros16 = jnp.zeros((16,), jnp.int32)
    neginf16 = jnp.full((16,), NEG_INF, jnp.float32)
    lane = _lane()

    def zero_hist():
        def zstep(i, _):
            hist[pl.ds(i * 16, 16)] = zeros16
            return 0
        lax.fori_loop(0, 256, zstep, 0)

    def walk(above0):
        def wstep(i, carry):
            above, bsel, found = carry
            d = 255 - i
            h = hist[pl.ds(d * 16, 16)]
            hs = jnp.sum(h)
            tot = above + hs
            hit = tot >= TKn
            take = jnp.logical_and(jnp.logical_not(found), hit)
            bsel = jnp.where(take, d, bsel)
            found_new = jnp.logical_or(found, hit)
            above = jnp.where(found_new, above, tot)
            return above, bsel, found_new
        above, bsel, _ = lax.fori_loop(
            0, 256, wstep, (above0, jnp.int32(0), jnp.bool_(False)))
        return above, bsel

    def spec_pass(row, t_f32):
        bufs = (buf, buf2)
        sems = (sem_a, sem_b)
        UNR = 4
        cps = {0: pltpu.async_copy(scores.at[row, pl.ds(0, CH)], buf, sem_a)}
        carry = (jnp.int32(0), jnp.bool_(False))
        for c in range(NCH):
            cur = bufs[c % 2]
            cps[c % 2].wait()
            if c + 1 < NCH:
                cps[(c + 1) % 2] = pltpu.async_copy(
                    scores.at[row, pl.ds((c + 1) * CH, CH)],
                    bufs[(c + 1) % 2], sems[(c + 1) % 2])

            def step(g, carry, cur=cur, c=c):
                coff, dropped = carry
                base = g * (16 * UNR)
                ss = [cur[pl.ds(base + k * 16, 16)] for k in range(UNR)]
                ms = [s >= t_f32 for s in ss]
                anym = ms[0]
                for k in range(1, UNR):
                    anym = jnp.logical_or(anym, ms[k])

                def hit(coff=coff, dropped=dropped, ss=ss, ms=ms,
                        base=base, c=c):
                    for k in range(UNR):
                        idx = (c * CH + base + k * 16) + lane
                        can = coff <= CAND - 16
                        mm = jnp.logical_and(ms[k], can)
                        plsc.store_compressed(
                            cnds.at[pl.ds(coff, 16)], ss[k], mask=mm)
                        plsc.store_compressed(
                            cndi.at[pl.ds(coff, 16)], idx, mask=mm)
                        coff = coff + _pop1(mm)
                        dropped = jnp.logical_or(
                            dropped, _pop1(ms[k]) > _pop1(mm))
                    return coff, dropped

                def miss(coff=coff, dropped=dropped):
                    return coff, dropped

                return lax.cond(jnp.any(anym), hit, miss)
            carry = lax.fori_loop(0, CH // (16 * UNR), step, carry)
        return carry

    def hist_row(row, shift, prefix, use_prefix):
        def chunk(c, _):
            pltpu.sync_copy(scores.at[row, pl.ds(c * CH, CH)], buf)
            def step(i, _):
                s = buf[pl.ds(i * 16, 16)]
                ku = _ku_of(s)
                dig = lax.shift_right_logical(ku, jnp.uint32(shift))
                dig = lax.bitcast_convert_type(dig, jnp.int32) & 0xFF
                slot = dig * 16 + lane
                if use_prefix:
                    hi = lax.shift_right_logical(ku, jnp.uint32(shift + 8))
                    m = hi == prefix
                    plsc.addupdate_scatter(hist, [slot], ones16, mask=m)
                else:
                    plsc.addupdate_scatter(hist, [slot], ones16)
                return 0
            lax.fori_loop(0, CH // 16, step, 0)
            return 0
        lax.fori_loop(0, NCH, chunk, 0)

    def compact_row(row, shift, pasm):
        def chunk(c, carry):
            pltpu.sync_copy(scores.at[row, pl.ds(c * CH, CH)], buf)
            def step(i, carry):
                topoff, coff, dropped = carry
                s = buf[pl.ds(i * 16, 16)]
                idx = (c * CH + i * 16) + lane
                ku = _ku_of(s)
                kus = lax.shift_right_logical(ku, jnp.uint32(shift))
                gt = kus > pasm
                eq = kus == pasm
                plsc.store_compressed(topv.at[pl.ds(topoff, 16)], s, mask=gt)
                plsc.store_compressed(topi.at[pl.ds(topoff, 16)], idx, mask=gt)
                topoff = topoff + _pop1(gt)
                can = coff <= CAND - 16
                eqm = jnp.logical_and(eq, can)
                plsc.store_compressed(cnds.at[pl.ds(coff, 16)], s, mask=eqm)
                plsc.store_compressed(cndi.at[pl.ds(coff, 16)], idx, mask=eqm)
                coff = coff + _pop1(eqm)
                dropped = jnp.logical_or(
                    dropped, _pop1(eq) > _pop1(eqm))
                return topoff, coff, dropped
            return lax.fori_loop(0, CH // 16, step, carry)
        return lax.fori_loop(
            0, NCH, chunk,
            (jnp.int32(0), jnp.int32(0), jnp.bool_(False)))

    def hist_cand(shift, coff):
        def step(i, _):
            valid = lane < (coff - i * 16)
            s = cnds[pl.ds(i * 16, 16)]
            ku = _ku_of(s)
            dig = lax.shift_right_logical(ku, jnp.uint32(shift))
            dig = lax.bitcast_convert_type(dig, jnp.int32) & 0xFF
            slot = dig * 16 + lane
            plsc.addupdate_scatter(hist, [slot], ones16, mask=valid)
            return 0
        n16 = (coff + 15) // 16
        lax.fori_loop(0, n16, step, 0)

    def compact_cand(shift, pasm, topoff0, coff0):
        def step(i, carry):
            topoff, coff = carry
            valid = lane < (coff0 - i * 16)
            s = cnds[pl.ds(i * 16, 16)]
            idx = cndi[pl.ds(i * 16, 16)]
            ku = _ku_of(s)
            kus = lax.shift_right_logical(ku, jnp.uint32(shift))
            gt = jnp.logical_and(valid, kus > pasm)
            eq = jnp.logical_and(valid, kus == pasm)
            plsc.store_compressed(topv.at[pl.ds(topoff, 16)], s, mask=gt)
            plsc.store_compressed(topi.at[pl.ds(topoff, 16)], idx, mask=gt)
            topoff = topoff + _pop1(gt)
            plsc.store_compressed(cnds2.at[pl.ds(coff, 16)], s, mask=eq)
            plsc.store_compressed(cndi2.at[pl.ds(coff, 16)], idx, mask=eq)
            coff = coff + _pop1(eq)
            return topoff, coff
        n16 = (coff0 + 15) // 16
        topoff, coff = lax.fori_loop(0, n16, step, (topoff0, jnp.int32(0)))
        def cstep(i, _):
            cnds[pl.ds(i * 16, 16)] = cnds2[pl.ds(i * 16, 16)]
            cndi[pl.ds(i * 16, 16)] = cndi2[pl.ds(i * 16, 16)]
            return 0
        lax.fori_loop(0, (coff + 15) // 16, cstep, 0)
        return topoff, coff

    def do_row(r, t_f32):
        row = wid * QT + r

        scoff, sdrop = spec_pass(row, t_f32)
        ok = jnp.logical_and(scoff >= TKn, jnp.logical_not(sdrop))

        zero_hist()

        def h_ok(scoff=scoff):
            hist_cand(24, scoff)
            return 0

        def h_fb(row=row):
            hist_row(row, 24, jnp.uint32(0), False)
            return 0

        lax.cond(ok, h_ok, h_fb)
        above, b = walk(jnp.int32(0))
        prefix = lax.convert_element_type(b, jnp.uint32)

        def c_ok(scoff=scoff, prefix=prefix):
            to, co = compact_cand(24, prefix, jnp.int32(0), scoff)
            return to, co, jnp.bool_(True)

        def c_fb(row=row, prefix=prefix):
            to, co, drp = compact_row(row, 24, prefix)
            return to, co, jnp.logical_not(drp)

        topoff, coff, compacted = lax.cond(ok, c_ok, c_fb)

        for lvl in (1, 2, 3):
            shift = 24 - 8 * lvl
            zero_hist()

            def hc(coff=coff, shift=shift):
                hist_cand(shift, coff)
                return 0

            def hr(row=row, shift=shift, prefix=prefix):
                hist_row(row, shift, prefix, True)
                return 0

            lax.cond(compacted, hc, hr)
            above, b = walk(above)
            pasm = (prefix << jnp.uint32(8)) | lax.convert_element_type(
                b, jnp.uint32)

            def cc(topoff=topoff, coff=coff, shift=shift, pasm=pasm):
                to, co = compact_cand(shift, pasm, topoff, coff)
                return to, co, jnp.bool_(True)

            def cr(row=row, shift=shift, pasm=pasm, lvl=lvl):
                to, co, drp = compact_row(row, shift, pasm)
                if lvl == 3:
                    return to, co, jnp.bool_(True)
                return to, co, jnp.logical_not(drp)

            topoff, coff, compacted = lax.cond(compacted, cc, cr)
            prefix = pasm

        def fstep(v, _):
            valid = lane < (topoff - v * 16)
            tv = topv[pl.ds(v * 16, 16)]
            ti = topi[pl.ds(v * 16, 16)]
            finv[pl.ds(v * 16, 16)] = jnp.where(valid, tv, neginf16)
            fini[pl.ds(v * 16, 16)] = jnp.where(valid, ti, zeros16)
            return 0
        lax.fori_loop(0, OUTW // 16, fstep, 0)

        def estep(v, off):
            valid = lane < (coff - v * 16)
            can = off <= OUTW - 16
            m = jnp.logical_and(valid, can)
            s = cnds[pl.ds(v * 16, 16)]
            idx = cndi[pl.ds(v * 16, 16)]
            plsc.store_compressed(finv.at[pl.ds(off, 16)], s, mask=m)
            plsc.store_compressed(fini.at[pl.ds(off, 16)], idx, mask=m)
            return off + _pop1(m)
        lax.fori_loop(0, OUTW // 16, estep, topoff)

        vs = [finv[pl.ds(v * 16, 16)] for v in range(OUTW // 16)]
        iss = [fini[pl.ds(v * 16, 16)] for v in range(OUTW // 16)]

        def ostep(ov, carry):
            vs, iss = carry

            def sstep(j, carry):
                vs, iss, ovec, oidx = carry
                vs = list(vs)
                mv = vs[0]
                for v in range(1, len(vs)):
                    mv = jnp.maximum(mv, vs[v])
                m = jnp.max(mv)
                iv = jnp.where(vs[0] == m, iss[0], I32_MAX)
                for v in range(1, len(vs)):
                    iv = jnp.minimum(
                        iv, jnp.where(vs[v] == m, iss[v], I32_MAX))
                sel = jnp.min(iv)
                hit = lane == j
                ovec = jnp.where(hit, m, ovec)
                oidx = jnp.where(hit, sel, oidx)
                for v in range(len(vs)):
                    rm = jnp.logical_and(vs[v] == m, iss[v] == sel)
                    vs[v] = jnp.where(rm, NEG_INF, vs[v])
                return tuple(vs), iss, ovec, oidx

            vs, iss, ovec, oidx = lax.fori_loop(
                0, 16, sstep,
                (vs, iss, jnp.zeros((16,), jnp.float32), zeros16))
            outvb[pl.ds(ov * 16, 16)] = ovec
            outib[pl.ds(ov * 16, 16)] = oidx
            return vs, iss
        lax.fori_loop(0, (TKn + 15) // 16, ostep, (tuple(vs), tuple(iss)))

        pltpu.sync_copy(outvb, outv.at[row])
        pltpu.sync_copy(outib, outi.at[row])
        last = outvb[pl.ds(96, 16)]
        g5 = outvb[pl.ds(80, 16)]
        s99 = last[TKn - 1 - 96]
        s80 = g5[0]
        return s99 - 2.0 * (s80 - s99)

    lax.fori_loop(0, QT, do_row, jnp.float32(jnp.inf))


@functools.partial(jax.jit, static_argnames=())
def _topk_sc(scores):
    mesh = plsc.VectorSubcoreMesh(core_axis_name="c", subcore_axis_name="s")
    f = pl.kernel(
        _topk_body,
        out_type=[
            jax.ShapeDtypeStruct((Qn, OUTW), jnp.float32),
            jax.ShapeDtypeStruct((Qn, OUTW), jnp.int32),
        ],
        mesh=mesh,
        compiler_params=pltpu.CompilerParams(needs_layout_passes=False),
        scratch_types=[
            pltpu.VMEM((CH,), jnp.float32),
            pltpu.VMEM((CH,), jnp.float32),
            pltpu.VMEM((CAND,), jnp.float32),
            pltpu.VMEM((CAND,), jnp.int32),
            pltpu.VMEM((CAND,), jnp.float32),
            pltpu.VMEM((CAND,), jnp.int32),
            pltpu.VMEM((4096,), jnp.int32),
            pltpu.VMEM((OUTW,), jnp.float32),
            pltpu.VMEM((OUTW,), jnp.int32),
            pltpu.VMEM((OUTW,), jnp.float32),
            pltpu.VMEM((OUTW,), jnp.int32),
            pltpu.VMEM((OUTW,), jnp.float32),
            pltpu.VMEM((OUTW,), jnp.int32),
            pltpu.SemaphoreType.DMA,
            pltpu.SemaphoreType.DMA,
        ],
    )
    return f(scores)


def kernel(input_image_emb, input_description_emb, W_fusion, b_fusion, keys):
    fused = pl.pallas_call(
        _fusion_body,
        out_shape=jax.ShapeDtypeStruct((Qn, Dn), jnp.float32),
    )(input_image_emb, input_description_emb, W_fusion,
      b_fusion.reshape(1, Dn))

    q = fused / (jnp.linalg.norm(fused, axis=-1, keepdims=True) + 1e-12)
    knorm = jnp.linalg.norm(keys, axis=-1, keepdims=True) + 1e-12

    keys_pad = jnp.pad(keys, ((0, KPAD - Kn), (0, 0)))
    knorm_pad = jnp.pad(knorm, ((0, KPAD - Kn), (0, 0)), constant_values=1.0)
    scores = pl.pallas_call(
        _scores_body,
        grid=(KPAD // KB,),
        in_specs=[
            pl.BlockSpec((Qn, Dn), lambda i: (0, 0)),
            pl.BlockSpec((KB, Dn), lambda i: (i, 0)),
            pl.BlockSpec((KB, 1), lambda i: (i, 0)),
        ],
        out_specs=pl.BlockSpec((Qn, KB), lambda i: (0, i)),
        out_shape=jax.ShapeDtypeStruct((Qn, KPAD), jnp.float32),
    )(q, keys_pad, knorm_pad)

    tv, ti = _topk_sc(scores)
    return tv[:, :TKn], ti[:, :TKn]


# speedup vs baseline: 3.8165x; 1.1704x over previous
import functools

import jax
import jax.numpy as jnp
from jax import lax
from jax.experimental import pallas as pl
from jax.experimental.pallas import tpu as pltpu
from jax.experimental.pallas import tpu_sc as plsc

Qn = 1024
Dn = 512
Kn = 100000
TKn = 100
KB = 1024
KPAD = 100352

NW = 32
QT = Qn // NW
CH = 12544
NCH = KPAD // CH
CAND = 12544
OUTW = 128
NEG_INF = float("-inf")
I32_MAX = 2147483647


def _fusion_body(img_ref, desc_ref, w_ref, b_ref, q_ref):
    concat = jnp.concatenate([img_ref[...], desc_ref[...]], axis=-1)
    q_ref[...] = jnp.tanh(jnp.dot(concat, w_ref[...]) + b_ref[...])


def _scores_body(q_ref, keys_ref, knorm_ref, out_ref):
    kn = keys_ref[...] / knorm_ref[...]
    s = lax.dot_general(q_ref[...], kn, (((1,), (1,)), ((), ())))
    pid = pl.program_id(0)
    col = pid * KB + lax.broadcasted_iota(jnp.int32, s.shape, 1)
    out_ref[...] = jnp.where(col < Kn, s, -jnp.inf)


def _lane():
    return lax.iota(jnp.int32, 16)


def _ku_of(s):
    bits = lax.bitcast_convert_type(s, jnp.int32)
    sign = lax.shift_right_arithmetic(bits, 31)
    kui = bits ^ (sign | jnp.int32(-(2 ** 31)))
    return lax.bitcast_convert_type(kui, jnp.uint32)


def _pop1(mask):
    return plsc.all_reduce_population_count(mask)[0]


def _topk_body(scores, outv, outi, buf, buf2, cnds, cndi, cnds2, cndi2, hist,
               topv, topi, finv, fini, outvb, outib, sem_a, sem_b):
    cid = lax.axis_index("c")
    sid = lax.axis_index("s")
    wid = sid * 2 + cid

    ones16 = jnp.full((16,), 1, jnp.int32)
    zeros16 = jnp.zeros((16,), jnp.int32)
    neginf16 = jnp.full((16,), NEG_INF, jnp.float32)
    lane = _lane()

    def zero_hist():
        def zstep(i, _):
            hist[pl.ds(i * 16, 16)] = zeros16
            return 0
        lax.fori_loop(0, 256, zstep, 0)

    def walk(above0):
        def wstep(i, carry):
            above, bsel, found = carry
            d = 255 - i
            h = hist[pl.ds(d * 16, 16)]
            hs = jnp.sum(h)
            tot = above + hs
            hit = tot >= TKn
            take = jnp.logical_and(jnp.logical_not(found), hit)
            bsel = jnp.where(take, d, bsel)
            found_new = jnp.logical_or(found, hit)
            above = jnp.where(found_new, above, tot)
            return above, bsel, found_new
        above, bsel, _ = lax.fori_loop(
            0, 256, wstep, (above0, jnp.int32(0), jnp.bool_(False)))
        return above, bsel

    def spec_pass(row, t_f32):
        bufs = (buf, buf2)
        sems = (sem_a, sem_b)
        UNR = 8
        cps = {0: pltpu.async_copy(scores.at[row, pl.ds(0, CH)], buf, sem_a)}
        carry = (jnp.int32(0), jnp.bool_(False))
        for c in range(NCH):
            cur = bufs[c % 2]
            cps[c % 2].wait()
            if c + 1 < NCH:
                cps[(c + 1) % 2] = pltpu.async_copy(
                    scores.at[row, pl.ds((c + 1) * CH, CH)],
                    bufs[(c + 1) % 2], sems[(c + 1) % 2])

            def step(g, carry, cur=cur, c=c):
                coff, dropped = carry
                base = g * (16 * UNR)
                ss = [cur[pl.ds(base + k * 16, 16)] for k in range(UNR)]
                mx = ss[0]
                for k in range(1, UNR):
                    mx = jnp.maximum(mx, ss[k])

                def hit(coff=coff, dropped=dropped, ss=ss, base=base, c=c):
                    for k in range(UNR):
                        m = ss[k] >= t_f32
                        idx = (c * CH + base + k * 16) + lane
                        can = coff <= CAND - 16
                        mm = jnp.logical_and(m, can)
                        plsc.store_compressed(
                            cnds.at[pl.ds(coff, 16)], ss[k], mask=mm)
                        plsc.store_compressed(
                            cndi.at[pl.ds(coff, 16)], idx, mask=mm)
                        coff = coff + _pop1(mm)
                        dropped = jnp.logical_or(
                            dropped, _pop1(m) > _pop1(mm))
                    return coff, dropped

                def miss(coff=coff, dropped=dropped):
                    return coff, dropped

                return lax.cond(jnp.max(mx) >= t_f32, hit, miss)
            carry = lax.fori_loop(0, CH // (16 * UNR), step, carry)
        return carry

    def hist_row(row, shift, prefix, use_prefix):
        def chunk(c, _):
            pltpu.sync_copy(scores.at[row, pl.ds(c * CH, CH)], buf)
            def step(i, _):
                s = buf[pl.ds(i * 16, 16)]
                ku = _ku_of(s)
                dig = lax.shift_right_logical(ku, jnp.uint32(shift))
                dig = lax.bitcast_convert_type(dig, jnp.int32) & 0xFF
                slot = dig * 16 + lane
                if use_prefix:
                    hi = lax.shift_right_logical(ku, jnp.uint32(shift + 8))
                    m = hi == prefix
                    plsc.addupdate_scatter(hist, [slot], ones16, mask=m)
                else:
                    plsc.addupdate_scatter(hist, [slot], ones16)
                return 0
            lax.fori_loop(0, CH // 16, step, 0)
            return 0
        lax.fori_loop(0, NCH, chunk, 0)

    def compact_row(row, shift, pasm):
        def chunk(c, carry):
            pltpu.sync_copy(scores.at[row, pl.ds(c * CH, CH)], buf)
            def step(i, carry):
                topoff, coff, dropped = carry
                s = buf[pl.ds(i * 16, 16)]
                idx = (c * CH + i * 16) + lane
                ku = _ku_of(s)
                kus = lax.shift_right_logical(ku, jnp.uint32(shift))
                gt = kus > pasm
                eq = kus == pasm
                plsc.store_compressed(topv.at[pl.ds(topoff, 16)], s, mask=gt)
                plsc.store_compressed(topi.at[pl.ds(topoff, 16)], idx, mask=gt)
                topoff = topoff + _pop1(gt)
                can = coff <= CAND - 16
                eqm = jnp.logical_and(eq, can)
                plsc.store_compressed(cnds.at[pl.ds(coff, 16)], s, mask=eqm)
                plsc.store_compressed(cndi.at[pl.ds(coff, 16)], idx, mask=eqm)
                coff = coff + _pop1(eqm)
                dropped = jnp.logical_or(
                    dropped, _pop1(eq) > _pop1(eqm))
                return topoff, coff, dropped
            return lax.fori_loop(0, CH // 16, step, carry)
        return lax.fori_loop(
            0, NCH, chunk,
            (jnp.int32(0), jnp.int32(0), jnp.bool_(False)))

    def hist_cand(shift, coff):
        def step(i, _):
            valid = lane < (coff - i * 16)
            s = cnds[pl.ds(i * 16, 16)]
            ku = _ku_of(s)
            dig = lax.shift_right_logical(ku, jnp.uint32(shift))
            dig = lax.bitcast_convert_type(dig, jnp.int32) & 0xFF
            slot = dig * 16 + lane
            plsc.addupdate_scatter(hist, [slot], ones16, mask=valid)
            return 0
        n16 = (coff + 15) // 16
        lax.fori_loop(0, n16, step, 0)

    def compact_cand(shift, pasm, topoff0, coff0):
        def step(i, carry):
            topoff, coff = carry
            valid = lane < (coff0 - i * 16)
            s = cnds[pl.ds(i * 16, 16)]
            idx = cndi[pl.ds(i * 16, 16)]
            ku = _ku_of(s)
            kus = lax.shift_right_logical(ku, jnp.uint32(shift))
            gt = jnp.logical_and(valid, kus > pasm)
            eq = jnp.logical_and(valid, kus == pasm)
            plsc.store_compressed(topv.at[pl.ds(topoff, 16)], s, mask=gt)
            plsc.store_compressed(topi.at[pl.ds(topoff, 16)], idx, mask=gt)
            topoff = topoff + _pop1(gt)
            plsc.store_compressed(cnds2.at[pl.ds(coff, 16)], s, mask=eq)
            plsc.store_compressed(cndi2.at[pl.ds(coff, 16)], idx, mask=eq)
            coff = coff + _pop1(eq)
            return topoff, coff
        n16 = (coff0 + 15) // 16
        topoff, coff = lax.fori_loop(0, n16, step, (topoff0, jnp.int32(0)))
        def cstep(i, _):
            cnds[pl.ds(i * 16, 16)] = cnds2[pl.ds(i * 16, 16)]
            cndi[pl.ds(i * 16, 16)] = cndi2[pl.ds(i * 16, 16)]
            return 0
        lax.fori_loop(0, (coff + 15) // 16, cstep, 0)
        return topoff, coff

    def do_row(r, t_f32):
        row = wid * QT + r

        scoff, sdrop = spec_pass(row, t_f32)
        ok = jnp.logical_and(scoff >= TKn, jnp.logical_not(sdrop))

        zero_hist()

        def h_ok(scoff=scoff):
            hist_cand(24, scoff)
            return 0

        def h_fb(row=row):
            hist_row(row, 24, jnp.uint32(0), False)
            return 0

        lax.cond(ok, h_ok, h_fb)
        above, b = walk(jnp.int32(0))
        prefix = lax.convert_element_type(b, jnp.uint32)

        def c_ok(scoff=scoff, prefix=prefix):
            to, co = compact_cand(24, prefix, jnp.int32(0), scoff)
            return to, co, jnp.bool_(True)

        def c_fb(row=row, prefix=prefix):
            to, co, drp = compact_row(row, 24, prefix)
            return to, co, jnp.logical_not(drp)

        topoff, coff, compacted = lax.cond(ok, c_ok, c_fb)

        for lvl in (1, 2, 3):
            shift = 24 - 8 * lvl
            zero_hist()

            def hc(coff=coff, shift=shift):
                hist_cand(shift, coff)
                return 0

            def hr(row=row, shift=shift, prefix=prefix):
                hist_row(row, shift, prefix, True)
                return 0

            lax.cond(compacted, hc, hr)
            above, b = walk(above)
            pasm = (prefix << jnp.uint32(8)) | lax.convert_element_type(
                b, jnp.uint32)

            def cc(topoff=topoff, coff=coff, shift=shift, pasm=pasm):
                to, co = compact_cand(shift, pasm, topoff, coff)
                return to, co, jnp.bool_(True)

            def cr(row=row, shift=shift, pasm=pasm, lvl=lvl):
                to, co, drp = compact_row(row, shift, pasm)
                if lvl == 3:
                    return to, co, jnp.bool_(True)
                return to, co, jnp.logical_not(drp)

            topoff, coff, compacted = lax.cond(compacted, cc, cr)
            prefix = pasm

        def fstep(v, _):
            valid = lane < (topoff - v * 16)
            tv = topv[pl.ds(v * 16, 16)]
            ti = topi[pl.ds(v * 16, 16)]
            finv[pl.ds(v * 16, 16)] = jnp.where(valid, tv, neginf16)
            fini[pl.ds(v * 16, 16)] = jnp.where(valid, ti, zeros16)
            return 0
        lax.fori_loop(0, OUTW // 16, fstep, 0)

        def estep(v, off):
            valid = lane < (coff - v * 16)
            can = off <= OUTW - 16
            m = jnp.logical_and(valid, can)
            s = cnds[pl.ds(v * 16, 16)]
            idx = cndi[pl.ds(v * 16, 16)]
            plsc.store_compressed(finv.at[pl.ds(off, 16)], s, mask=m)
            plsc.store_compressed(fini.at[pl.ds(off, 16)], idx, mask=m)
            return off + _pop1(m)
        lax.fori_loop(0, OUTW // 16, estep, topoff)

        vs = [finv[pl.ds(v * 16, 16)] for v in range(OUTW // 16)]
        iss = [fini[pl.ds(v * 16, 16)] for v in range(OUTW // 16)]

        def ostep(ov, carry):
            vs, iss = carry

            def sstep(j, carry):
                vs, iss, ovec, oidx = carry
                vs = list(vs)
                mv = vs[0]
                for v in range(1, len(vs)):
                    mv = jnp.maximum(mv, vs[v])
                m = jnp.max(mv)
                iv = jnp.where(vs[0] == m, iss[0], I32_MAX)
                for v in range(1, len(vs)):
                    iv = jnp.minimum(
                        iv, jnp.where(vs[v] == m, iss[v], I32_MAX))
                sel = jnp.min(iv)
                hit = lane == j
                ovec = jnp.where(hit, m, ovec)
                oidx = jnp.where(hit, sel, oidx)
                for v in range(len(vs)):
                    rm = jnp.logical_and(vs[v] == m, iss[v] == sel)
                    vs[v] = jnp.where(rm, NEG_INF, vs[v])
                return tuple(vs), iss, ovec, oidx

            vs, iss, ovec, oidx = lax.fori_loop(
                0, 16, sstep,
                (vs, iss, jnp.zeros((16,), jnp.float32), zeros16))
            outvb[pl.ds(ov * 16, 16)] = ovec
            outib[pl.ds(ov * 16, 16)] = oidx
            return vs, iss
        lax.fori_loop(0, (TKn + 15) // 16, ostep, (tuple(vs), tuple(iss)))

        pltpu.sync_copy(outvb, outv.at[row])
        pltpu.sync_copy(outib, outi.at[row])
        last = outvb[pl.ds(96, 16)]
        g5 = outvb[pl.ds(80, 16)]
        s99 = last[TKn - 1 - 96]
        s80 = g5[0]
        return s99 - 2.0 * (s80 - s99)

    lax.fori_loop(0, QT, do_row, jnp.float32(jnp.inf))


@functools.partial(jax.jit, static_argnames=())
def _topk_sc(scores):
    mesh = plsc.VectorSubcoreMesh(core_axis_name="c", subcore_axis_name="s")
    f = pl.kernel(
        _topk_body,
        out_type=[
            jax.ShapeDtypeStruct((Qn, OUTW), jnp.float32),
            jax.ShapeDtypeStruct((Qn, OUTW), jnp.int32),
        ],
        mesh=mesh,
        compiler_params=pltpu.CompilerParams(needs_layout_passes=False),
        scratch_types=[
            pltpu.VMEM((CH,), jnp.float32),
            pltpu.VMEM((CH,), jnp.float32),
            pltpu.VMEM((CAND,), jnp.float32),
            pltpu.VMEM((CAND,), jnp.int32),
            pltpu.VMEM((CAND,), jnp.float32),
            pltpu.VMEM((CAND,), jnp.int32),
            pltpu.VMEM((4096,), jnp.int32),
            pltpu.VMEM((OUTW,), jnp.float32),
            pltpu.VMEM((OUTW,), jnp.int32),
            pltpu.VMEM((OUTW,), jnp.float32),
            pltpu.VMEM((OUTW,), jnp.int32),
            pltpu.VMEM((OUTW,), jnp.float32),
            pltpu.VMEM((OUTW,), jnp.int32),
            pltpu.SemaphoreType.DMA,
            pltpu.SemaphoreType.DMA,
        ],
    )
    return f(scores)


def kernel(input_image_emb, input_description_emb, W_fusion, b_fusion, keys):
    fused = pl.pallas_call(
        _fusion_body,
        out_shape=jax.ShapeDtypeStruct((Qn, Dn), jnp.float32),
    )(input_image_emb, input_description_emb, W_fusion,
      b_fusion.reshape(1, Dn))

    q = fused / (jnp.linalg.norm(fused, axis=-1, keepdims=True) + 1e-12)
    knorm = jnp.linalg.norm(keys, axis=-1, keepdims=True) + 1e-12

    keys_pad = jnp.pad(keys, ((0, KPAD - Kn), (0, 0)))
    knorm_pad = jnp.pad(knorm, ((0, KPAD - Kn), (0, 0)), constant_values=1.0)
    scores = pl.pallas_call(
        _scores_body,
        grid=(KPAD // KB,),
        in_specs=[
            pl.BlockSpec((Qn, Dn), lambda i: (0, 0)),
            pl.BlockSpec((KB, Dn), lambda i: (i, 0)),
            pl.BlockSpec((KB, 1), lambda i: (i, 0)),
        ],
        out_specs=pl.BlockSpec((Qn, KB), lambda i: (0, i)),
        out_shape=jax.ShapeDtypeStruct((Qn, KPAD), jnp.float32),
    )(q, keys_pad, knorm_pad)

    tv, ti = _topk_sc(scores)
    return tv[:, :TKn], ti[:, :TKn]


# speedup vs baseline: 3.8244x; 1.0021x over previous
import functools

import jax
import jax.numpy as jnp
from jax import lax
from jax.experimental import pallas as pl
from jax.experimental.pallas import tpu as pltpu
from jax.experimental.pallas import tpu_sc as plsc

Qn = 1024
Dn = 512
Kn = 100000
TKn = 100
KB = 1024
KPAD = 100352

NW = 32
QT = Qn // NW
CH = 12544
NCH = KPAD // CH
CAND = 12544
OUTW = 128
NEG_INF = float("-inf")
I32_MAX = 2147483647


def _fusion_body(img_ref, desc_ref, w_ref, b_ref, q_ref):
    concat = jnp.concatenate([img_ref[...], desc_ref[...]], axis=-1)
    q_ref[...] = jnp.tanh(jnp.dot(concat, w_ref[...]) + b_ref[...])


def _scores_body(q_ref, keys_ref, knorm_ref, out_ref):
    kn = keys_ref[...] / knorm_ref[...]
    s = lax.dot_general(q_ref[...], kn, (((1,), (1,)), ((), ())))
    pid = pl.program_id(0)
    col = pid * KB + lax.broadcasted_iota(jnp.int32, s.shape, 1)
    out_ref[...] = jnp.where(col < Kn, s, -jnp.inf)


def _lane():
    return lax.iota(jnp.int32, 16)


def _ku_of(s):
    bits = lax.bitcast_convert_type(s, jnp.int32)
    sign = lax.shift_right_arithmetic(bits, 31)
    kui = bits ^ (sign | jnp.int32(-(2 ** 31)))
    return lax.bitcast_convert_type(kui, jnp.uint32)


def _pop1(mask):
    return plsc.all_reduce_population_count(mask)[0]


def _topk_body(scores, outv, outi, buf, buf2, cnds, cndi, cnds2, cndi2, hist,
               topv, topi, finv, fini, outvb, outib, sem_a, sem_b):
    cid = lax.axis_index("c")
    sid = lax.axis_index("s")
    wid = sid * 2 + cid

    ones16 = jnp.full((16,), 1, jnp.int32)
    zeros16 = jnp.zeros((16,), jnp.int32)
    neginf16 = jnp.full((16,), NEG_INF, jnp.float32)
    lane = _lane()

    def zero_hist():
        def zstep(i, _):
            hist[pl.ds(i * 16, 16)] = zeros16
            return 0
        lax.fori_loop(0, 256, zstep, 0)

    def walk(above0):
        def wstep(i, carry):
            above, bsel, found = carry
            d = 255 - i
            h = hist[pl.ds(d * 16, 16)]
            hs = jnp.sum(h)
            tot = above + hs
            hit = tot >= TKn
            take = jnp.logical_and(jnp.logical_not(found), hit)
            bsel = jnp.where(take, d, bsel)
            found_new = jnp.logical_or(found, hit)
            above = jnp.where(found_new, above, tot)
            return above, bsel, found_new
        above, bsel, _ = lax.fori_loop(
            0, 256, wstep, (above0, jnp.int32(0), jnp.bool_(False)))
        return above, bsel

    def spec_pass(row, t_f32):
        bufs = (buf, buf2)
        sems = (sem_a, sem_b)
        UNR = 8
        cps = {0: pltpu.async_copy(scores.at[row, pl.ds(0, CH)], buf, sem_a)}
        carry = (jnp.int32(0), jnp.bool_(False))
        for c in range(NCH):
            cur = bufs[c % 2]
            cps[c % 2].wait()
            if c + 1 < NCH:
                cps[(c + 1) % 2] = pltpu.async_copy(
                    scores.at[row, pl.ds((c + 1) * CH, CH)],
                    bufs[(c + 1) % 2], sems[(c + 1) % 2])

            def step(g, carry, cur=cur, c=c):
                coff, dropped = carry
                base = g * (16 * UNR)
                ss = [cur[pl.ds(base + k * 16, 16)] for k in range(UNR)]
                mx = ss[0]
                for k in range(1, UNR):
                    mx = jnp.maximum(mx, ss[k])

                def hit(coff=coff, dropped=dropped, ss=ss, base=base, c=c):
                    for k in range(UNR):
                        m = ss[k] >= t_f32
                        idx = (c * CH + base + k * 16) + lane
                        can = coff <= CAND - 16
                        mm = jnp.logical_and(m, can)
                        plsc.store_compressed(
                            cnds.at[pl.ds(coff, 16)], ss[k], mask=mm)
                        plsc.store_compressed(
                            cndi.at[pl.ds(coff, 16)], idx, mask=mm)
                        coff = coff + _pop1(mm)
                        dropped = jnp.logical_or(
                            dropped, _pop1(m) > _pop1(mm))
                    return coff, dropped

                def miss(coff=coff, dropped=dropped):
                    return coff, dropped

                return lax.cond(jnp.max(mx) >= t_f32, hit, miss)
            carry = lax.fori_loop(0, CH // (16 * UNR), step, carry)
        return carry

    def hist_row(row, shift, prefix, use_prefix):
        def chunk(c, _):
            pltpu.sync_copy(scores.at[row, pl.ds(c * CH, CH)], buf)
            def step(i, _):
                s = buf[pl.ds(i * 16, 16)]
                ku = _ku_of(s)
                dig = lax.shift_right_logical(ku, jnp.uint32(shift))
                dig = lax.bitcast_convert_type(dig, jnp.int32) & 0xFF
                slot = dig * 16 + lane
                if use_prefix:
                    hi = lax.shift_right_logical(ku, jnp.uint32(shift + 8))
                    m = hi == prefix
                    plsc.addupdate_scatter(hist, [slot], ones16, mask=m)
                else:
                    plsc.addupdate_scatter(hist, [slot], ones16)
                return 0
            lax.fori_loop(0, CH // 16, step, 0)
            return 0
        lax.fori_loop(0, NCH, chunk, 0)

    def compact_row(row, shift, pasm):
        def chunk(c, carry):
            pltpu.sync_copy(scores.at[row, pl.ds(c * CH, CH)], buf)
            def step(i, carry):
                topoff, coff, dropped = carry
                s = buf[pl.ds(i * 16, 16)]
                idx = (c * CH + i * 16) + lane
                ku = _ku_of(s)
                kus = lax.shift_right_logical(ku, jnp.uint32(shift))
                gt = kus > pasm
                eq = kus == pasm
                plsc.store_compressed(topv.at[pl.ds(topoff, 16)], s, mask=gt)
                plsc.store_compressed(topi.at[pl.ds(topoff, 16)], idx, mask=gt)
                topoff = topoff + _pop1(gt)
                can = coff <= CAND - 16
                eqm = jnp.logical_and(eq, can)
                plsc.store_compressed(cnds.at[pl.ds(coff, 16)], s, mask=eqm)
                plsc.store_compressed(cndi.at[pl.ds(coff, 16)], idx, mask=eqm)
                coff = coff + _pop1(eqm)
                dropped = jnp.logical_or(
                    dropped, _pop1(eq) > _pop1(eqm))
                return topoff, coff, dropped
            return lax.fori_loop(0, CH // 16, step, carry)
        return lax.fori_loop(
            0, NCH, chunk,
            (jnp.int32(0), jnp.int32(0), jnp.bool_(False)))

    def hist_cand(shift, coff):
        def step(i, _):
            valid = lane < (coff - i * 16)
            s = cnds[pl.ds(i * 16, 16)]
            ku = _ku_of(s)
            dig = lax.shift_right_logical(ku, jnp.uint32(shift))
            dig = lax.bitcast_convert_type(dig, jnp.int32) & 0xFF
            slot = dig * 16 + lane
            plsc.addupdate_scatter(hist, [slot], ones16, mask=valid)
            return 0
        n16 = (coff + 15) // 16
        lax.fori_loop(0, n16, step, 0)

    def compact_cand(shift, pasm, topoff0, coff0):
        def step(i, carry):
            topoff, coff = carry
            valid = lane < (coff0 - i * 16)
            s = cnds[pl.ds(i * 16, 16)]
            idx = cndi[pl.ds(i * 16, 16)]
            ku = _ku_of(s)
            kus = lax.shift_right_logical(ku, jnp.uint32(shift))
            gt = jnp.logical_and(valid, kus > pasm)
            eq = jnp.logical_and(valid, kus == pasm)
            plsc.store_compressed(topv.at[pl.ds(topoff, 16)], s, mask=gt)
            plsc.store_compressed(topi.at[pl.ds(topoff, 16)], idx, mask=gt)
            topoff = topoff + _pop1(gt)
            plsc.store_compressed(cnds2.at[pl.ds(coff, 16)], s, mask=eq)
            plsc.store_compressed(cndi2.at[pl.ds(coff, 16)], idx, mask=eq)
            coff = coff + _pop1(eq)
            return topoff, coff
        n16 = (coff0 + 15) // 16
        topoff, coff = lax.fori_loop(0, n16, step, (topoff0, jnp.int32(0)))
        def cstep(i, _):
            cnds[pl.ds(i * 16, 16)] = cnds2[pl.ds(i * 16, 16)]
            cndi[pl.ds(i * 16, 16)] = cndi2[pl.ds(i * 16, 16)]
            return 0
        lax.fori_loop(0, (coff + 15) // 16, cstep, 0)
        return topoff, coff

    def do_row(r, t_f32):
        row = wid * QT + r

        scoff, sdrop = spec_pass(row, t_f32)
        ok = jnp.logical_and(scoff >= TKn, jnp.logical_not(sdrop))

        zero_hist()

        def h_ok(scoff=scoff):
            hist_cand(24, scoff)
            return 0

        def h_fb(row=row):
            hist_row(row, 24, jnp.uint32(0), False)
            return 0

        lax.cond(ok, h_ok, h_fb)
        above, b = walk(jnp.int32(0))
        prefix = lax.convert_element_type(b, jnp.uint32)

        def c_ok(scoff=scoff, prefix=prefix):
            to, co = compact_cand(24, prefix, jnp.int32(0), scoff)
            return to, co, jnp.bool_(True)

        def c_fb(row=row, prefix=prefix):
            to, co, drp = compact_row(row, 24, prefix)
            return to, co, jnp.logical_not(drp)

        topoff, coff, compacted = lax.cond(ok, c_ok, c_fb)

        for lvl in (1, 2, 3):
            shift = 24 - 8 * lvl
            zero_hist()

            def hc(coff=coff, shift=shift):
                hist_cand(shift, coff)
                return 0

            def hr(row=row, shift=shift, prefix=prefix):
                hist_row(row, shift, prefix, True)
                return 0

            lax.cond(compacted, hc, hr)
            above, b = walk(above)
            pasm = (prefix << jnp.uint32(8)) | lax.convert_element_type(
                b, jnp.uint32)

            def cc(topoff=topoff, coff=coff, shift=shift, pasm=pasm):
                to, co = compact_cand(shift, pasm, topoff, coff)
                return to, co, jnp.bool_(True)

            def cr(row=row, shift=shift, pasm=pasm, lvl=lvl):
                to, co, drp = compact_row(row, shift, pasm)
                if lvl == 3:
                    return to, co, jnp.bool_(True)
                return to, co, jnp.logical_not(drp)

            topoff, coff, compacted = lax.cond(compacted, cc, cr)
            prefix = pasm

        def fstep(v, _):
            valid = lane < (topoff - v * 16)
            tv = topv[pl.ds(v * 16, 16)]
            ti = topi[pl.ds(v * 16, 16)]
            finv[pl.ds(v * 16, 16)] = jnp.where(valid, tv, neginf16)
            fini[pl.ds(v * 16, 16)] = jnp.where(valid, ti, zeros16)
            return 0
        lax.fori_loop(0, OUTW // 16, fstep, 0)

        def estep(v, off):
            valid = lane < (coff - v * 16)
            can = off <= OUTW - 16
            m = jnp.logical_and(valid, can)
            s = cnds[pl.ds(v * 16, 16)]
            idx = cndi[pl.ds(v * 16, 16)]
            plsc.store_compressed(finv.at[pl.ds(off, 16)], s, mask=m)
            plsc.store_compressed(fini.at[pl.ds(off, 16)], idx, mask=m)
            return off + _pop1(m)
        lax.fori_loop(0, OUTW // 16, estep, topoff)

        vs = [finv[pl.ds(v * 16, 16)] for v in range(OUTW // 16)]
        iss = [fini[pl.ds(v * 16, 16)] for v in range(OUTW // 16)]

        def ostep(ov, carry):
            vs, iss = carry

            def sstep(j, carry):
                vs, iss, ovec, oidx = carry
                vs = list(vs)
                mv = vs[0]
                for v in range(1, len(vs)):
                    mv = jnp.maximum(mv, vs[v])
                m = jnp.max(mv)
                iv = jnp.where(vs[0] == m, iss[0], I32_MAX)
                for v in range(1, len(vs)):
                    iv = jnp.minimum(
                        iv, jnp.where(vs[v] == m, iss[v], I32_MAX))
                sel = jnp.min(iv)
                hit = lane == j
                ovec = jnp.where(hit, m, ovec)
                oidx = jnp.where(hit, sel, oidx)
                for v in range(len(vs)):
                    rm = jnp.logical_and(vs[v] == m, iss[v] == sel)
                    vs[v] = jnp.where(rm, NEG_INF, vs[v])
                return tuple(vs), iss, ovec, oidx

            vs, iss, ovec, oidx = lax.fori_loop(
                0, 16, sstep,
                (vs, iss, jnp.zeros((16,), jnp.float32), zeros16))
            outvb[pl.ds(ov * 16, 16)] = ovec
            outib[pl.ds(ov * 16, 16)] = oidx
            return vs, iss
        lax.fori_loop(0, (TKn + 15) // 16, ostep, (tuple(vs), tuple(iss)))

        pltpu.sync_copy(outvb, outv.at[row])
        pltpu.sync_copy(outib, outi.at[row])
        last = outvb[pl.ds(96, 16)]
        g5 = outvb[pl.ds(80, 16)]
        s99 = last[TKn - 1 - 96]
        s80 = g5[0]
        return s99 - 2.0 * (s80 - s99)

    lax.fori_loop(0, QT, do_row, jnp.float32(jnp.inf))


@functools.partial(jax.jit, static_argnames=())
def _topk_sc(scores):
    mesh = plsc.VectorSubcoreMesh(core_axis_name="c", subcore_axis_name="s")
    f = pl.kernel(
        _topk_body,
        out_type=[
            jax.ShapeDtypeStruct((Qn, OUTW), jnp.float32),
            jax.ShapeDtypeStruct((Qn, OUTW), jnp.int32),
        ],
        mesh=mesh,
        compiler_params=pltpu.CompilerParams(needs_layout_passes=False),
        scratch_types=[
            pltpu.VMEM((CH,), jnp.float32),
            pltpu.VMEM((CH,), jnp.float32),
            pltpu.VMEM((CAND,), jnp.float32),
            pltpu.VMEM((CAND,), jnp.int32),
            pltpu.VMEM((CAND,), jnp.float32),
            pltpu.VMEM((CAND,), jnp.int32),
            pltpu.VMEM((4096,), jnp.int32),
            pltpu.VMEM((OUTW,), jnp.float32),
            pltpu.VMEM((OUTW,), jnp.int32),
            pltpu.VMEM((OUTW,), jnp.float32),
            pltpu.VMEM((OUTW,), jnp.int32),
            pltpu.VMEM((OUTW,), jnp.float32),
            pltpu.VMEM((OUTW,), jnp.int32),
            pltpu.SemaphoreType.DMA,
            pltpu.SemaphoreType.DMA,
        ],
    )
    return f(scores)


def kernel(input_image_emb, input_description_emb, W_fusion, b_fusion, keys):
    concat = jnp.concatenate([input_image_emb, input_description_emb],
                             axis=-1)
    fused = jnp.tanh(concat @ W_fusion + b_fusion)
    q = fused / (jnp.linalg.norm(fused, axis=-1, keepdims=True) + 1e-12)
    knorm = jnp.linalg.norm(keys, axis=-1, keepdims=True) + 1e-12

    keys_pad = jnp.pad(keys, ((0, KPAD - Kn), (0, 0)))
    knorm_pad = jnp.pad(knorm, ((0, KPAD - Kn), (0, 0)), constant_values=1.0)
    scores = pl.pallas_call(
        _scores_body,
        grid=(KPAD // KB,),
        in_specs=[
            pl.BlockSpec((Qn, Dn), lambda i: (0, 0)),
            pl.BlockSpec((KB, Dn), lambda i: (i, 0)),
            pl.BlockSpec((KB, 1), lambda i: (i, 0)),
        ],
        out_specs=pl.BlockSpec((Qn, KB), lambda i: (0, i)),
        out_shape=jax.ShapeDtypeStruct((Qn, KPAD), jnp.float32),
    )(q, keys_pad, knorm_pad)

    tv, ti = _topk_sc(scores)
    return tv[:, :TKn], ti[:, :TKn]
